# baseline (speedup 1.0000x reference)
import functools

import jax
import jax.numpy as jnp
from jax import lax
from jax.experimental import pallas as pl
from jax.experimental.pallas import tpu as pltpu
from jax.experimental.pallas import tpu_sc as plsc

N = 10000
E = 320000
D = 128
H = 32
NC = 2
NS = 16
NW = NC * NS
EPW = E // NW
K = 100
NB = EPW // K
RPT = N // NS
DEGW = 16
_DEGF = 8
KA = 100
NBA = EPW // KA

_sc_mesh = functools.partial(
    plsc.VectorSubcoreMesh, core_axis_name="c", subcore_axis_name="s")
_sc_params = pltpu.CompilerParams(use_tc_tiling_on_sc=False)


def _sc_degree(dstR, zdeg):
  @functools.partial(
      pl.kernel,
      out_type=jax.ShapeDtypeStruct((NC, N, DEGW), jnp.float32),
      mesh=_sc_mesh(),
      compiler_params=_sc_params,
      scratch_types=[
          pltpu.VMEM((NB, K), jnp.int32),
          pltpu.VMEM((K, DEGW), jnp.float32),
          pltpu.VMEM_SHARED((N, DEGW), jnp.float32),
          pltpu.SemaphoreType.DMA,
          pltpu.SemaphoreType.DMA,
      ],
  )
  def deg_kernel(dst_hbm, z_hbm, out_hbm, dstbuf, ones, acc, zsem, isem):
    c = lax.axis_index("c")
    s = lax.axis_index("s")
    wid = s * NC + c
    zcp = pltpu.async_copy(z_hbm.at[pl.ds(s * RPT, RPT)],
                           acc.at[pl.ds(s * RPT, RPT)], zsem)
    icp = pltpu.async_copy(dst_hbm.at[wid], dstbuf, isem)
    for i in range(K):
      ones[i, :] = jnp.full((DEGW,), 1.0, jnp.float32)
    zcp.wait()
    icp.wait()
    plsc.subcore_barrier()

    for j in range(_DEGF):
      pltpu.async_copy(ones, acc.at[dstbuf.at[j]], isem, add=True)

    def body(i, _):
      pltpu.make_async_copy(ones, acc.at[dstbuf.at[i]], isem).wait()
      pltpu.async_copy(ones, acc.at[dstbuf.at[i + _DEGF]], isem, add=True)
      return ()

    lax.fori_loop(0, NB - _DEGF, body, ())
    for j in range(_DEGF):
      pltpu.make_async_copy(ones, acc.at[dstbuf.at[NB - _DEGF + j]],
                            isem).wait()
    plsc.subcore_barrier()
    pltpu.sync_copy(acc.at[pl.ds(s * RPT, RPT)],
                    out_hbm.at[c, pl.ds(s * RPT, RPT)])

  return deg_kernel(dstR, zdeg)


def _sc_aggregate(y, srcR, dstR, z2d):
  @functools.partial(
      pl.kernel,
      out_type=jax.ShapeDtypeStruct((NC, N, D), jnp.float32),
      mesh=_sc_mesh(),
      compiler_params=_sc_params,
      scratch_types=[
          pltpu.VMEM((NBA, KA), jnp.int32),
          pltpu.VMEM((NBA, KA), jnp.int32),
          pltpu.VMEM((KA, D), jnp.float32),
          pltpu.VMEM((KA, D), jnp.float32),
          pltpu.SemaphoreType.DMA,
          pltpu.SemaphoreType.DMA,
          pltpu.VMEM_SHARED((N, D), jnp.float32),
      ],
  )
  def agg_kernel(y_hbm, src_hbm, dst_hbm, z_hbm, out_hbm,
                 srcbuf, dstbuf, rows0, rows1, sem0, sem1, acc):
    c = lax.axis_index("c")
    s = lax.axis_index("s")
    wid = s * NC + c
    zcp = pltpu.async_copy(z_hbm.at[pl.ds(s * RPT, RPT)],
                           acc.at[pl.ds(s * RPT, RPT)], sem0)
    scp = pltpu.async_copy(src_hbm.at[wid], srcbuf, sem1)
    dcp = pltpu.async_copy(dst_hbm.at[wid], dstbuf, sem1)
    zcp.wait()
    scp.wait()
    dcp.wait()
    plsc.subcore_barrier()

    pltpu.async_copy(y_hbm.at[srcbuf.at[0]], rows0, sem0)

    def body(i, _):
      b0 = 2 * i
      b1 = 2 * i + 1
      b2 = lax.rem(b0 + 2, NBA)
      pltpu.async_copy(y_hbm.at[srcbuf.at[b1]], rows1, sem1)
      pltpu.make_async_copy(y_hbm.at[srcbuf.at[b0]], rows0, sem0).wait()
      pltpu.sync_copy(rows0, acc.at[dstbuf.at[b0]], add=True)
      pltpu.async_copy(y_hbm.at[srcbuf.at[b2]], rows0, sem0)
      pltpu.make_async_copy(y_hbm.at[srcbuf.at[b1]], rows1, sem1).wait()
      pltpu.sync_copy(rows1, acc.at[dstbuf.at[b1]], add=True)
      return ()

    lax.fori_loop(0, NBA // 2, body, ())
    pltpu.make_async_copy(y_hbm.at[srcbuf.at[0]], rows0, sem0).wait()
    plsc.subcore_barrier()
    pltpu.sync_copy(acc.at[pl.ds(s * RPT, RPT)],
                    out_hbm.at[c, pl.ds(s * RPT, RPT)])

  return agg_kernel(y, srcR, dstR, z2d)


_NBLK = 2
_BR = N // _NBLK
_HBLK = 5
_HBR = N // _HBLK


def _row_spec(width):
  return pl.BlockSpec((_BR, width), lambda i: (i, 0))


def _full_spec(shape):
  nd = len(shape)
  return pl.BlockSpec(shape, lambda i: (0,) * nd)


def _tc_xw1(state, W1):
  def body(st_ref, w_ref, o_ref):
    o_ref[...] = jnp.dot(st_ref[...], w_ref[...],
                         preferred_element_type=jnp.float32,
                         precision=lax.Precision.HIGHEST)

  return pl.pallas_call(
      body,
      grid=(_NBLK,),
      in_specs=[_row_spec(D), _full_spec((D, D))],
      out_specs=[_row_spec(D)],
      out_shape=[jax.ShapeDtypeStruct((N, D), jnp.float32)],
  )(state, W1)[0]


def _tc_prep(degP, xw1):
  def body(dp_ref, xw_ref, dis_ref, y_ref):
    dp = dp_ref[...]
    deg = dp[0][:, 0:1] + dp[1][:, 0:1] + 1.0
    dis = 1.0 / jnp.sqrt(deg)
    dis_ref[...] = dis
    y_ref[...] = dis * xw_ref[...]

  return pl.pallas_call(
      body,
      grid=(_NBLK,),
      in_specs=[
          pl.BlockSpec((NC, _BR, DEGW), lambda i: (0, i, 0)),
          _row_spec(D),
      ],
      out_specs=[_row_spec(1), _row_spec(D)],
      out_shape=[
          jax.ShapeDtypeStruct((N, 1), jnp.float32),
          jax.ShapeDtypeStruct((N, D), jnp.float32),
      ],
  )(degP, xw1)


def _tc_layer(P, y, dis, b, Wn):
  def body(p_ref, y_ref, dis_ref, b_ref, w_ref, x_ref, yn_ref):
    p = p_ref[...]
    dis = dis_ref[...]
    x = jax.nn.relu(dis * (p[0] + p[1] + y_ref[...]) + b_ref[...])
    x_ref[...] = x
    yn_ref[...] = dis * jnp.dot(x, w_ref[...],
                                preferred_element_type=jnp.float32,
                                precision=lax.Precision.HIGHEST)

  return pl.pallas_call(
      body,
      grid=(_NBLK,),
      in_specs=[
          pl.BlockSpec((NC, _BR, D), lambda i: (0, i, 0)),
          _row_spec(D),
          _row_spec(1),
          _full_spec((1, D)),
          _full_spec((D, D)),
      ],
      out_specs=[_row_spec(D), _row_spec(D)],
      out_shape=[
          jax.ShapeDtypeStruct((N, D), jnp.float32),
          jax.ShapeDtypeStruct((N, D), jnp.float32),
      ],
  )(P, y, dis, b, Wn)


def _tc_layer_last(P, y, dis, b):
  def body(p_ref, y_ref, dis_ref, b_ref, x_ref):
    p = p_ref[...]
    x_ref[...] = jax.nn.relu(
        dis_ref[...] * (p[0] + p[1] + y_ref[...]) + b_ref[...])

  return pl.pallas_call(
      body,
      grid=(_NBLK,),
      in_specs=[
          pl.BlockSpec((NC, _BR, D), lambda i: (0, i, 0)),
          _row_spec(D),
          _row_spec(1),
          _full_spec((1, D)),
      ],
      out_specs=[_row_spec(D)],
      out_shape=[jax.ShapeDtypeStruct((N, D), jnp.float32)],
  )(P, y, dis, b)


def _tc_head(xs, state, lw1r, lb1, lw2, lb2, lw3, lb3):
  def body(x1_ref, x2_ref, x3_ref, x4_ref, x5_ref, st_ref, w1_ref, b1_ref,
           w2_ref, b2_ref, w3_ref, b3_ref, act_ref, conc_scr, sum_scr):
    p = pl.program_id(0)
    j = pl.program_id(1)

    @pl.when(p == 0)
    def _():
      w1 = w1_ref[...]
      h = b1_ref[...]
      ins = (x1_ref, x2_ref, x3_ref, x4_ref, x5_ref, st_ref)
      for k in range(6):
        h = h + jnp.dot(ins[k][...], w1[k],
                        preferred_element_type=jnp.float32,
                        precision=lax.Precision.HIGHEST)
      h = jnp.where(h >= 0, h, 0.01 * h)
      h = jnp.dot(h, w2_ref[...],
                  precision=lax.Precision.HIGHEST) + b2_ref[...]
      h = jnp.where(h >= 0, h, 0.01 * h)
      cc = jnp.dot(h, w3_ref[...],
                   precision=lax.Precision.HIGHEST) + b3_ref[...]
      cblk = jax.nn.softplus(cc)
      conc_scr[pl.ds(j * _HBR, _HBR), :] = cblk

      @pl.when(j == 0)
      def _():
        sum_scr[...] = jnp.zeros_like(sum_scr)

      sum_scr[...] = sum_scr[...] + jnp.sum(cblk).reshape(1, 1)

    @pl.when(p == 1)
    def _():
      act_ref[...] = (conc_scr[pl.ds(j * _HBR, _HBR), :]
                      / (sum_scr[...] + 1e-20))

  row_in = pl.BlockSpec((_HBR, D), lambda p, jj: (jnp.where(p == 0, jj, 0), 0))
  return pl.pallas_call(
      body,
      grid=(2, _HBLK),
      in_specs=[
          row_in, row_in, row_in, row_in, row_in, row_in,
          pl.BlockSpec((6, D, H), lambda p, jj: (0, 0, 0)),
          pl.BlockSpec((1, H), lambda p, jj: (0, 0)),
          pl.BlockSpec((H, H), lambda p, jj: (0, 0)),
          pl.BlockSpec((1, H), lambda p, jj: (0, 0)),
          pl.BlockSpec((H, 1), lambda p, jj: (0, 0)),
          pl.BlockSpec((1, 1), lambda p, jj: (0, 0)),
      ],
      out_specs=[pl.BlockSpec((_HBR, 1), lambda p, jj: (jj, 0))],
      out_shape=[jax.ShapeDtypeStruct((N, 1), jnp.float32)],
      scratch_shapes=[
          pltpu.VMEM((N, 1), jnp.float32),
          pltpu.VMEM((1, 1), jnp.float32),
      ],
  )(*xs, state, lw1r, lb1, lw2, lb2, lw3, lb3)[0]


def kernel(state, edge_index, W1, b1, W2, b2, W3, b3, W4, b4, W5, b5,
           lw1, lb1, lw2, lb2, lw3, lb3):
  srcR = edge_index[0].reshape(NW, NBA, KA)
  dstR = edge_index[1].reshape(NW, NBA, KA)
  dstRdeg = edge_index[1].reshape(NW, NB, K)
  z2d = jnp.zeros((N, D), jnp.float32)
  zdeg = jnp.zeros((N, DEGW), jnp.float32)

  xw1 = _tc_xw1(state, W1)
  degP = _sc_degree(dstRdeg, zdeg)
  dis, y = _tc_prep(degP, xw1)

  biases = (b1, b2, b3, b3, b3)
  nexts = (W2, W3, W3, W3, None)
  xs = []
  for k in range(5):
    P = _sc_aggregate(y, srcR, dstR, z2d)
    if nexts[k] is not None:
      x, y = _tc_layer(P, y, dis, biases[k].reshape(1, D), nexts[k])
    else:
      x = _tc_layer_last(P, y, dis, biases[k].reshape(1, D))[0]
    xs.append(x)

  action = _tc_head(xs, state, lw1.reshape(6, D, H), lb1.reshape(1, H),
                    lw2, lb2.reshape(1, H), lw3, lb3.reshape(1, 1))
  return action.reshape(N)


# speedup vs baseline: 1.0084x; 1.0084x over previous
import functools

import jax
import jax.numpy as jnp
from jax import lax
from jax.experimental import pallas as pl
from jax.experimental.pallas import tpu as pltpu
from jax.experimental.pallas import tpu_sc as plsc

N = 10000
E = 320000
D = 128
H = 32
NC = 2
NS = 16
NW = NC * NS
EPW = E // NW
K = 100
NB = EPW // K
RPT = N // NS
DEGW = 16
_DEGF = 8
KA = 100
NBA = EPW // KA

_sc_mesh = functools.partial(
    plsc.VectorSubcoreMesh, core_axis_name="c", subcore_axis_name="s")
_sc_params = pltpu.CompilerParams(use_tc_tiling_on_sc=False)


def _sc_degree(dstR, zdeg):
  @functools.partial(
      pl.kernel,
      out_type=jax.ShapeDtypeStruct((NC, N, DEGW), jnp.float32),
      mesh=_sc_mesh(),
      compiler_params=_sc_params,
      scratch_types=[
          pltpu.VMEM((NB, K), jnp.int32),
          pltpu.VMEM((K, DEGW), jnp.float32),
          pltpu.VMEM_SHARED((N, DEGW), jnp.float32),
          pltpu.SemaphoreType.DMA,
          pltpu.SemaphoreType.DMA,
      ],
  )
  def deg_kernel(dst_hbm, z_hbm, out_hbm, dstbuf, ones, acc, zsem, isem):
    c = lax.axis_index("c")
    s = lax.axis_index("s")
    wid = s * NC + c
    zcp = pltpu.async_copy(z_hbm.at[pl.ds(s * RPT, RPT)],
                           acc.at[pl.ds(s * RPT, RPT)], zsem)
    icp = pltpu.async_copy(dst_hbm.at[wid], dstbuf, isem)
    for i in range(K):
      ones[i, :] = jnp.full((DEGW,), 1.0, jnp.float32)
    zcp.wait()
    icp.wait()
    plsc.subcore_barrier()

    for j in range(_DEGF):
      pltpu.async_copy(ones, acc.at[dstbuf.at[j]], isem, add=True)

    def body(i, _):
      pltpu.make_async_copy(ones, acc.at[dstbuf.at[i]], isem).wait()
      pltpu.async_copy(ones, acc.at[dstbuf.at[i + _DEGF]], isem, add=True)
      return ()

    lax.fori_loop(0, NB - _DEGF, body, ())
    for j in range(_DEGF):
      pltpu.make_async_copy(ones, acc.at[dstbuf.at[NB - _DEGF + j]],
                            isem).wait()
    plsc.subcore_barrier()
    pltpu.sync_copy(acc.at[pl.ds(s * RPT, RPT)],
                    out_hbm.at[c, pl.ds(s * RPT, RPT)])

  return deg_kernel(dstR, zdeg)


def _sc_aggregate(y, srcR, dstR, z2d):
  @functools.partial(
      pl.kernel,
      out_type=jax.ShapeDtypeStruct((NC, N, D), jnp.float32),
      mesh=_sc_mesh(),
      compiler_params=_sc_params,
      scratch_types=[
          pltpu.VMEM((NBA, KA), jnp.int32),
          pltpu.VMEM((NBA, KA), jnp.int32),
          pltpu.VMEM((KA, D), jnp.float32),
          pltpu.VMEM((KA, D), jnp.float32),
          pltpu.SemaphoreType.DMA,
          pltpu.SemaphoreType.DMA,
          pltpu.VMEM_SHARED((N, D), jnp.float32),
      ],
  )
  def agg_kernel(y_hbm, src_hbm, dst_hbm, z_hbm, out_hbm,
                 srcbuf, dstbuf, rows0, rows1, sem0, sem1, acc):
    c = lax.axis_index("c")
    s = lax.axis_index("s")
    wid = s * NC + c
    zcp = pltpu.async_copy(z_hbm.at[pl.ds(s * RPT, RPT)],
                           acc.at[pl.ds(s * RPT, RPT)], sem0)
    scp = pltpu.async_copy(src_hbm.at[wid], srcbuf, sem1)
    dcp = pltpu.async_copy(dst_hbm.at[wid], dstbuf, sem1)
    zcp.wait()
    scp.wait()
    dcp.wait()
    plsc.subcore_barrier()

    pltpu.async_copy(y_hbm.at[srcbuf.at[0]], rows0, sem0)

    def body(i, _):
      b0 = 2 * i
      b1 = 2 * i + 1
      b2 = lax.rem(b0 + 2, NBA)
      pltpu.async_copy(y_hbm.at[srcbuf.at[b1]], rows1, sem1)
      pltpu.make_async_copy(y_hbm.at[srcbuf.at[b0]], rows0, sem0).wait()
      pltpu.sync_copy(rows0, acc.at[dstbuf.at[b0]], add=True)
      pltpu.async_copy(y_hbm.at[srcbuf.at[b2]], rows0, sem0)
      pltpu.make_async_copy(y_hbm.at[srcbuf.at[b1]], rows1, sem1).wait()
      pltpu.sync_copy(rows1, acc.at[dstbuf.at[b1]], add=True)
      return ()

    lax.fori_loop(0, NBA // 2, body, ())
    pltpu.make_async_copy(y_hbm.at[srcbuf.at[0]], rows0, sem0).wait()
    plsc.subcore_barrier()
    pltpu.sync_copy(acc.at[pl.ds(s * RPT, RPT)],
                    out_hbm.at[c, pl.ds(s * RPT, RPT)])

  return agg_kernel(y, srcR, dstR, z2d)


_NBLK = 5
_BR = N // _NBLK
_HBLK = 5
_HBR = N // _HBLK


def _row_spec(width):
  return pl.BlockSpec((_BR, width), lambda i: (i, 0))


def _full_spec(shape):
  nd = len(shape)
  return pl.BlockSpec(shape, lambda i: (0,) * nd)


def _tc_xw1(state, W1):
  def body(st_ref, w_ref, o_ref):
    o_ref[...] = jnp.dot(st_ref[...], w_ref[...],
                         preferred_element_type=jnp.float32,
                         precision=lax.Precision.HIGHEST)

  return pl.pallas_call(
      body,
      grid=(_NBLK,),
      in_specs=[_row_spec(D), _full_spec((D, D))],
      out_specs=[_row_spec(D)],
      out_shape=[jax.ShapeDtypeStruct((N, D), jnp.float32)],
  )(state, W1)[0]


def _tc_prep(degP, xw1):
  def body(dp_ref, xw_ref, dis_ref, y_ref):
    dp = dp_ref[...]
    deg = dp[0][:, 0:1] + dp[1][:, 0:1] + 1.0
    dis = 1.0 / jnp.sqrt(deg)
    dis_ref[...] = dis
    y_ref[...] = dis * xw_ref[...]

  return pl.pallas_call(
      body,
      grid=(_NBLK,),
      in_specs=[
          pl.BlockSpec((NC, _BR, DEGW), lambda i: (0, i, 0)),
          _row_spec(D),
      ],
      out_specs=[_row_spec(1), _row_spec(D)],
      out_shape=[
          jax.ShapeDtypeStruct((N, 1), jnp.float32),
          jax.ShapeDtypeStruct((N, D), jnp.float32),
      ],
  )(degP, xw1)


def _tc_layer(P, y, dis, b, Wn):
  def body(p_ref, y_ref, dis_ref, b_ref, w_ref, x_ref, yn_ref):
    p = p_ref[...]
    dis = dis_ref[...]
    x = jax.nn.relu(dis * (p[0] + p[1] + y_ref[...]) + b_ref[...])
    x_ref[...] = x
    yn_ref[...] = dis * jnp.dot(x, w_ref[...],
                                preferred_element_type=jnp.float32,
                                precision=lax.Precision.HIGHEST)

  return pl.pallas_call(
      body,
      grid=(_NBLK,),
      in_specs=[
          pl.BlockSpec((NC, _BR, D), lambda i: (0, i, 0)),
          _row_spec(D),
          _row_spec(1),
          _full_spec((1, D)),
          _full_spec((D, D)),
      ],
      out_specs=[_row_spec(D), _row_spec(D)],
      out_shape=[
          jax.ShapeDtypeStruct((N, D), jnp.float32),
          jax.ShapeDtypeStruct((N, D), jnp.float32),
      ],
  )(P, y, dis, b, Wn)


def _tc_layer_last(P, y, dis, b):
  def body(p_ref, y_ref, dis_ref, b_ref, x_ref):
    p = p_ref[...]
    x_ref[...] = jax.nn.relu(
        dis_ref[...] * (p[0] + p[1] + y_ref[...]) + b_ref[...])

  return pl.pallas_call(
      body,
      grid=(_NBLK,),
      in_specs=[
          pl.BlockSpec((NC, _BR, D), lambda i: (0, i, 0)),
          _row_spec(D),
          _row_spec(1),
          _full_spec((1, D)),
      ],
      out_specs=[_row_spec(D)],
      out_shape=[jax.ShapeDtypeStruct((N, D), jnp.float32)],
  )(P, y, dis, b)


def _tc_head(xs, state, lw1r, lb1, lw2, lb2, lw3, lb3):
  def body(x1_ref, x2_ref, x3_ref, x4_ref, x5_ref, st_ref, w1_ref, b1_ref,
           w2_ref, b2_ref, w3_ref, b3_ref, act_ref, conc_scr, sum_scr):
    p = pl.program_id(0)
    j = pl.program_id(1)

    @pl.when(p == 0)
    def _():
      w1 = w1_ref[...]
      h = b1_ref[...]
      ins = (x1_ref, x2_ref, x3_ref, x4_ref, x5_ref, st_ref)
      for k in range(6):
        h = h + jnp.dot(ins[k][...], w1[k],
                        preferred_element_type=jnp.float32,
                        precision=lax.Precision.HIGHEST)
      h = jnp.where(h >= 0, h, 0.01 * h)
      h = jnp.dot(h, w2_ref[...],
                  precision=lax.Precision.HIGHEST) + b2_ref[...]
      h = jnp.where(h >= 0, h, 0.01 * h)
      cc = jnp.dot(h, w3_ref[...],
                   precision=lax.Precision.HIGHEST) + b3_ref[...]
      cblk = jax.nn.softplus(cc)
      conc_scr[pl.ds(j * _HBR, _HBR), :] = cblk

      @pl.when(j == 0)
      def _():
        sum_scr[...] = jnp.zeros_like(sum_scr)

      sum_scr[...] = sum_scr[...] + jnp.sum(cblk).reshape(1, 1)

    @pl.when(p == 1)
    def _():
      act_ref[...] = (conc_scr[pl.ds(j * _HBR, _HBR), :]
                      / (sum_scr[...] + 1e-20))

  row_in = pl.BlockSpec((_HBR, D), lambda p, jj: (jnp.where(p == 0, jj, 0), 0))
  return pl.pallas_call(
      body,
      grid=(2, _HBLK),
      in_specs=[
          row_in, row_in, row_in, row_in, row_in, row_in,
          pl.BlockSpec((6, D, H), lambda p, jj: (0, 0, 0)),
          pl.BlockSpec((1, H), lambda p, jj: (0, 0)),
          pl.BlockSpec((H, H), lambda p, jj: (0, 0)),
          pl.BlockSpec((1, H), lambda p, jj: (0, 0)),
          pl.BlockSpec((H, 1), lambda p, jj: (0, 0)),
          pl.BlockSpec((1, 1), lambda p, jj: (0, 0)),
      ],
      out_specs=[pl.BlockSpec((_HBR, 1), lambda p, jj: (jj, 0))],
      out_shape=[jax.ShapeDtypeStruct((N, 1), jnp.float32)],
      scratch_shapes=[
          pltpu.VMEM((N, 1), jnp.float32),
          pltpu.VMEM((1, 1), jnp.float32),
      ],
  )(*xs, state, lw1r, lb1, lw2, lb2, lw3, lb3)[0]


def kernel(state, edge_index, W1, b1, W2, b2, W3, b3, W4, b4, W5, b5,
           lw1, lb1, lw2, lb2, lw3, lb3):
  srcR = edge_index[0].reshape(NW, NBA, KA)
  dstR = edge_index[1].reshape(NW, NBA, KA)
  dstRdeg = edge_index[1].reshape(NW, NB, K)
  z2d = jnp.zeros((N, D), jnp.float32)
  zdeg = jnp.zeros((N, DEGW), jnp.float32)

  xw1 = _tc_xw1(state, W1)
  degP = _sc_degree(dstRdeg, zdeg)
  dis, y = _tc_prep(degP, xw1)

  biases = (b1, b2, b3, b3, b3)
  nexts = (W2, W3, W3, W3, None)
  xs = []
  for k in range(5):
    P = _sc_aggregate(y, srcR, dstR, z2d)
    if nexts[k] is not None:
      x, y = _tc_layer(P, y, dis, biases[k].reshape(1, D), nexts[k])
    else:
      x = _tc_layer_last(P, y, dis, biases[k].reshape(1, D))[0]
    xs.append(x)

  action = _tc_head(xs, state, lw1.reshape(6, D, H), lb1.reshape(1, H),
                    lw2, lb2.reshape(1, H), lw3, lb3.reshape(1, 1))
  return action.reshape(N)


# speedup vs baseline: 1.0242x; 1.0157x over previous
import functools

import jax
import jax.numpy as jnp
from jax import lax
from jax.experimental import pallas as pl
from jax.experimental.pallas import tpu as pltpu
from jax.experimental.pallas import tpu_sc as plsc

N = 10000
E = 320000
D = 128
H = 32
NC = 2
NS = 16
NW = NC * NS
EPW = E // NW
K = 100
NB = EPW // K
RPT = N // NS
DEGW = 16
_DEGF = 8
KA = 100
NBA = EPW // KA

_sc_mesh = functools.partial(
    plsc.VectorSubcoreMesh, core_axis_name="c", subcore_axis_name="s")
_sc_params = pltpu.CompilerParams(use_tc_tiling_on_sc=False)


def _sc_degree(dstR, zdeg):
  @functools.partial(
      pl.kernel,
      out_type=jax.ShapeDtypeStruct((NC, N, DEGW), jnp.float32),
      mesh=_sc_mesh(),
      compiler_params=_sc_params,
      scratch_types=[
          pltpu.VMEM((NB, K), jnp.int32),
          pltpu.VMEM((K, DEGW), jnp.float32),
          pltpu.VMEM_SHARED((N, DEGW), jnp.float32),
          pltpu.SemaphoreType.DMA,
          pltpu.SemaphoreType.DMA,
      ],
  )
  def deg_kernel(dst_hbm, z_hbm, out_hbm, dstbuf, ones, acc, zsem, isem):
    c = lax.axis_index("c")
    s = lax.axis_index("s")
    wid = s * NC + c
    zcp = pltpu.async_copy(z_hbm.at[pl.ds(s * RPT, RPT)],
                           acc.at[pl.ds(s * RPT, RPT)], zsem)
    icp = pltpu.async_copy(dst_hbm.at[wid], dstbuf, isem)
    for i in range(K):
      ones[i, :] = jnp.full((DEGW,), 1.0, jnp.float32)
    zcp.wait()
    icp.wait()
    plsc.subcore_barrier()

    for j in range(_DEGF):
      pltpu.async_copy(ones, acc.at[dstbuf.at[j]], isem, add=True)

    def body(i, _):
      pltpu.make_async_copy(ones, acc.at[dstbuf.at[i]], isem).wait()
      pltpu.async_copy(ones, acc.at[dstbuf.at[i + _DEGF]], isem, add=True)
      return ()

    lax.fori_loop(0, NB - _DEGF, body, ())
    for j in range(_DEGF):
      pltpu.make_async_copy(ones, acc.at[dstbuf.at[NB - _DEGF + j]],
                            isem).wait()
    plsc.subcore_barrier()
    pltpu.sync_copy(acc.at[pl.ds(s * RPT, RPT)],
                    out_hbm.at[c, pl.ds(s * RPT, RPT)])

  return deg_kernel(dstR, zdeg)


def _sc_aggregate(y, srcR, dstR, z2d):
  @functools.partial(
      pl.kernel,
      out_type=jax.ShapeDtypeStruct((NC, N, D), jnp.float32),
      mesh=_sc_mesh(),
      compiler_params=_sc_params,
      scratch_types=[
          pltpu.VMEM((NBA, KA), jnp.int32),
          pltpu.VMEM((NBA, KA), jnp.int32),
          pltpu.VMEM((KA, D), jnp.float32),
          pltpu.VMEM((KA, D), jnp.float32),
          pltpu.SemaphoreType.DMA,
          pltpu.SemaphoreType.DMA,
          pltpu.VMEM_SHARED((N, D), jnp.float32),
      ],
  )
  def agg_kernel(y_hbm, src_hbm, dst_hbm, z_hbm, out_hbm,
                 srcbuf, dstbuf, rows0, rows1, sem0, sem1, acc):
    c = lax.axis_index("c")
    s = lax.axis_index("s")
    wid = s * NC + c
    zcp = pltpu.async_copy(z_hbm.at[pl.ds(s * RPT, RPT)],
                           acc.at[pl.ds(s * RPT, RPT)], sem0)
    scp = pltpu.async_copy(src_hbm.at[wid], srcbuf, sem1)
    dcp = pltpu.async_copy(dst_hbm.at[wid], dstbuf, sem1)
    zcp.wait()
    scp.wait()
    dcp.wait()
    plsc.subcore_barrier()

    pltpu.async_copy(y_hbm.at[srcbuf.at[0]], rows0, sem0)

    def body(i, _):
      b0 = 2 * i
      b1 = 2 * i + 1
      b2 = lax.rem(b0 + 2, NBA)
      pltpu.async_copy(y_hbm.at[srcbuf.at[b1]], rows1, sem1)
      pltpu.make_async_copy(y_hbm.at[srcbuf.at[b0]], rows0, sem0).wait()
      pltpu.sync_copy(rows0, acc.at[dstbuf.at[b0]], add=True)
      pltpu.async_copy(y_hbm.at[srcbuf.at[b2]], rows0, sem0)
      pltpu.make_async_copy(y_hbm.at[srcbuf.at[b1]], rows1, sem1).wait()
      pltpu.sync_copy(rows1, acc.at[dstbuf.at[b1]], add=True)
      return ()

    lax.fori_loop(0, NBA // 2, body, ())
    pltpu.make_async_copy(y_hbm.at[srcbuf.at[0]], rows0, sem0).wait()
    plsc.subcore_barrier()
    pltpu.sync_copy(acc.at[pl.ds(s * RPT, RPT)],
                    out_hbm.at[c, pl.ds(s * RPT, RPT)])

  return agg_kernel(y, srcR, dstR, z2d)


_NBLK = 5
_BR = N // _NBLK
_HBLK = 5
_HBR = N // _HBLK


def _row_spec(width):
  return pl.BlockSpec((_BR, width), lambda i: (i, 0))


def _full_spec(shape):
  nd = len(shape)
  return pl.BlockSpec(shape, lambda i: (0,) * nd)


def _tc_xw1(state, W1, lw15, lb1):
  def body(st_ref, w_ref, l_ref, b_ref, o_ref, h_ref):
    st = st_ref[...]
    o_ref[...] = jnp.dot(st, w_ref[...],
                         preferred_element_type=jnp.float32,
                         precision=lax.Precision.HIGHEST)
    h_ref[...] = jnp.dot(st, l_ref[...],
                         preferred_element_type=jnp.float32,
                         precision=lax.Precision.HIGHEST) + b_ref[...]

  return pl.pallas_call(
      body,
      grid=(_NBLK,),
      in_specs=[_row_spec(D), _full_spec((D, D)), _full_spec((D, H)),
                _full_spec((1, H))],
      out_specs=[_row_spec(D), _row_spec(H)],
      out_shape=[
          jax.ShapeDtypeStruct((N, D), jnp.float32),
          jax.ShapeDtypeStruct((N, H), jnp.float32),
      ],
  )(state, W1, lw15, lb1)


def _tc_prep(degP, xw1):
  def body(dp_ref, xw_ref, dis_ref, y_ref):
    dp = dp_ref[...]
    deg = dp[0][:, 0:1] + dp[1][:, 0:1] + 1.0
    dis = 1.0 / jnp.sqrt(deg)
    dis_ref[...] = dis
    y_ref[...] = dis * xw_ref[...]

  return pl.pallas_call(
      body,
      grid=(_NBLK,),
      in_specs=[
          pl.BlockSpec((NC, _BR, DEGW), lambda i: (0, i, 0)),
          _row_spec(D),
      ],
      out_specs=[_row_spec(1), _row_spec(D)],
      out_shape=[
          jax.ShapeDtypeStruct((N, 1), jnp.float32),
          jax.ShapeDtypeStruct((N, D), jnp.float32),
      ],
  )(degP, xw1)


def _tc_layer(P, y, dis, b, Wn, lw1k, hin):
  def body(p_ref, y_ref, dis_ref, b_ref, w_ref, l_ref, h_ref,
           yn_ref, ho_ref):
    p = p_ref[...]
    dis = dis_ref[...]
    x = jax.nn.relu(dis * (p[0] + p[1] + y_ref[...]) + b_ref[...])
    yn_ref[...] = dis * jnp.dot(x, w_ref[...],
                                preferred_element_type=jnp.float32,
                                precision=lax.Precision.HIGHEST)
    ho_ref[...] = h_ref[...] + jnp.dot(x, l_ref[...],
                                       preferred_element_type=jnp.float32,
                                       precision=lax.Precision.HIGHEST)

  return pl.pallas_call(
      body,
      grid=(_NBLK,),
      in_specs=[
          pl.BlockSpec((NC, _BR, D), lambda i: (0, i, 0)),
          _row_spec(D),
          _row_spec(1),
          _full_spec((1, D)),
          _full_spec((D, D)),
          _full_spec((D, H)),
          _row_spec(H),
      ],
      out_specs=[_row_spec(D), _row_spec(H)],
      out_shape=[
          jax.ShapeDtypeStruct((N, D), jnp.float32),
          jax.ShapeDtypeStruct((N, H), jnp.float32),
      ],
  )(P, y, dis, b, Wn, lw1k, hin)


def _tc_layer_last(P, y, dis, b, lw1k, hin):
  def body(p_ref, y_ref, dis_ref, b_ref, l_ref, h_ref, ho_ref):
    p = p_ref[...]
    x = jax.nn.relu(
        dis_ref[...] * (p[0] + p[1] + y_ref[...]) + b_ref[...])
    ho_ref[...] = h_ref[...] + jnp.dot(x, l_ref[...],
                                       preferred_element_type=jnp.float32,
                                       precision=lax.Precision.HIGHEST)

  return pl.pallas_call(
      body,
      grid=(_NBLK,),
      in_specs=[
          pl.BlockSpec((NC, _BR, D), lambda i: (0, i, 0)),
          _row_spec(D),
          _row_spec(1),
          _full_spec((1, D)),
          _full_spec((D, H)),
          _row_spec(H),
      ],
      out_specs=[_row_spec(H)],
      out_shape=[jax.ShapeDtypeStruct((N, H), jnp.float32)],
  )(P, y, dis, b, lw1k, hin)


def _tc_head(h, lw2, lb2, lw3, lb3):
  def body(h_ref, w2_ref, b2_ref, w3_ref, b3_ref, act_ref,
           conc_scr, sum_scr):
    p = pl.program_id(0)
    j = pl.program_id(1)

    @pl.when(p == 0)
    def _():
      hh = h_ref[...]
      hh = jnp.where(hh >= 0, hh, 0.01 * hh)
      hh = jnp.dot(hh, w2_ref[...],
                   precision=lax.Precision.HIGHEST) + b2_ref[...]
      hh = jnp.where(hh >= 0, hh, 0.01 * hh)
      cc = jnp.dot(hh, w3_ref[...],
                   precision=lax.Precision.HIGHEST) + b3_ref[...]
      cblk = jax.nn.softplus(cc)
      conc_scr[pl.ds(j * _HBR, _HBR), :] = cblk

      @pl.when(j == 0)
      def _():
        sum_scr[...] = jnp.zeros_like(sum_scr)

      sum_scr[...] = sum_scr[...] + jnp.sum(cblk).reshape(1, 1)

    @pl.when(p == 1)
    def _():
      act_ref[...] = (conc_scr[pl.ds(j * _HBR, _HBR), :]
                      / (sum_scr[...] + 1e-20))

  row_in = pl.BlockSpec((_HBR, H), lambda p, jj: (jnp.where(p == 0, jj, 0), 0))
  return pl.pallas_call(
      body,
      grid=(2, _HBLK),
      in_specs=[
          row_in,
          pl.BlockSpec((H, H), lambda p, jj: (0, 0)),
          pl.BlockSpec((1, H), lambda p, jj: (0, 0)),
          pl.BlockSpec((H, 1), lambda p, jj: (0, 0)),
          pl.BlockSpec((1, 1), lambda p, jj: (0, 0)),
      ],
      out_specs=[pl.BlockSpec((_HBR, 1), lambda p, jj: (jj, 0))],
      out_shape=[jax.ShapeDtypeStruct((N, 1), jnp.float32)],
      scratch_shapes=[
          pltpu.VMEM((N, 1), jnp.float32),
          pltpu.VMEM((1, 1), jnp.float32),
      ],
  )(h, lw2, lb2, lw3, lb3)[0]


def kernel(state, edge_index, W1, b1, W2, b2, W3, b3, W4, b4, W5, b5,
           lw1, lb1, lw2, lb2, lw3, lb3):
  srcR = edge_index[0].reshape(NW, NBA, KA)
  dstR = edge_index[1].reshape(NW, NBA, KA)
  dstRdeg = edge_index[1].reshape(NW, NB, K)
  z2d = jnp.zeros((N, D), jnp.float32)
  zdeg = jnp.zeros((N, DEGW), jnp.float32)

  lw1r = lw1.reshape(6, D, H)
  xw1, h = _tc_xw1(state, W1, lw1r[5], lb1.reshape(1, H))
  degP = _sc_degree(dstRdeg, zdeg)
  dis, y = _tc_prep(degP, xw1)

  biases = (b1, b2, b3, b3, b3)
  nexts = (W2, W3, W3, W3, None)
  for k in range(5):
    P = _sc_aggregate(y, srcR, dstR, z2d)
    if nexts[k] is not None:
      y, h = _tc_layer(P, y, dis, biases[k].reshape(1, D), nexts[k],
                       lw1r[k], h)
    else:
      h = _tc_layer_last(P, y, dis, biases[k].reshape(1, D),
                         lw1r[k], h)[0]

  action = _tc_head(h, lw2, lb2.reshape(1, H), lw3, lb3.reshape(1, 1))
  return action.reshape(N)
